# baseline (speedup 1.0000x reference)
import functools

import jax
import jax.numpy as jnp
from jax import lax
from jax.experimental import pallas as pl
from jax.experimental.pallas import tpu as pltpu
from jax.experimental.pallas import tpu_sc as plsc

B = 4096
L = 50
T = 20
D = 32

NC = 2
NS = 16
NW = NC * NS

IN_PER_W = B * L // NW
TGT_PER_W = B * T // NW
IN_IDX_ROWS = IN_PER_W // 128
TGT_IDX_ROWS = TGT_PER_W // 128
K = 5


def _sc_gather(emb_u, emb_v, inp_idx, tgt_idx):
    mesh = plsc.VectorSubcoreMesh(core_axis_name="c", subcore_axis_name="s")

    @functools.partial(
        pl.kernel,
        mesh=mesh,
        compiler_params=pltpu.CompilerParams(use_tc_tiling_on_sc=False),
        out_type=[
            jax.ShapeDtypeStruct((NW, IN_PER_W, D), jnp.float32),
            jax.ShapeDtypeStruct((NW, TGT_PER_W, D), jnp.float32),
        ],
        scratch_types=[
            pltpu.VMEM((IN_IDX_ROWS, 128), jnp.int32),
            pltpu.VMEM((TGT_IDX_ROWS, 128), jnp.int32),
            pltpu.VMEM((K * 128, D), jnp.float32),
            pltpu.SemaphoreType.DMA,
        ],
    )
    def k(emb_u_hbm, emb_v_hbm, iidx_hbm, tidx_hbm, irows_hbm, trows_hbm,
          iidx_v, tidx_v, buf_v, sem):
        wid = lax.axis_index("s") * NC + lax.axis_index("c")
        pltpu.sync_copy(iidx_hbm.at[wid], iidx_v)
        pltpu.sync_copy(tidx_hbm.at[wid], tidx_v)
        my_irows = irows_hbm.at[wid]
        my_trows = trows_hbm.at[wid]

        @pl.loop(0, IN_IDX_ROWS // K)
        def _(g):
            cps = [
                pltpu.async_copy(
                    emb_u_hbm.at[iidx_v.at[g * K + j]],
                    buf_v.at[pl.ds(j * 128, 128)], sem)
                for j in range(K)
            ]
            for c in cps:
                c.wait()
            pltpu.sync_copy(buf_v, my_irows.at[pl.ds(g * (K * 128), K * 128)])

        @pl.loop(0, TGT_IDX_ROWS // K)
        def _(g):
            cps = [
                pltpu.async_copy(
                    emb_v_hbm.at[tidx_v.at[g * K + j]],
                    buf_v.at[pl.ds(j * 128, 128)], sem)
                for j in range(K)
            ]
            for c in cps:
                c.wait()
            pltpu.sync_copy(buf_v, my_trows.at[pl.ds(g * (K * 128), K * 128)])

    return k(emb_u, emb_v, inp_idx, tgt_idx)


BBLK = 128


def _tc_loss_body(irows_ref, trows_ref, tgt0_ref, out_ref):
    pid = pl.program_id(0)
    x = irows_ref[...].reshape(BBLK, L, D)
    ctx = jnp.mean(x, axis=1)
    tgt = trows_ref[...].reshape(BBLK, T, D)
    logits = jnp.sum(ctx[:, None, :] * tgt, axis=-1)
    sig = jax.nn.sigmoid(logits)
    bits = lax.broadcasted_iota(jnp.int32, (BBLK, T), 1)
    codes = ((tgt0_ref[...] >> bits) & 1).astype(jnp.float32)
    p = codes * sig + (1.0 - codes) * (1.0 - sig)
    part = -jnp.sum(jnp.log(p + 1e-9))

    @pl.when(pid == 0)
    def _():
        out_ref[0, 0] = 0.0

    out_ref[0, 0] += part


def _tc_loss(inp_rows, tgt_rows, tgt0):
    grid = B // BBLK
    return pl.pallas_call(
        _tc_loss_body,
        grid=(grid,),
        in_specs=[
            pl.BlockSpec((BBLK * L, D), lambda i: (i, 0)),
            pl.BlockSpec((BBLK * T, D), lambda i: (i, 0)),
            pl.BlockSpec((BBLK, 1), lambda i: (i, 0)),
        ],
        out_specs=pl.BlockSpec(memory_space=pltpu.MemorySpace.SMEM),
        out_shape=jax.ShapeDtypeStruct((1, 1), jnp.float32),
    )(inp_rows, tgt_rows, tgt0)


def kernel(input, target, vocabs, emb_u, emb_v):
    inp_idx = input.reshape(NW, IN_IDX_ROWS, 128).astype(jnp.int32)
    tgt_idx = target.reshape(NW, TGT_IDX_ROWS, 128).astype(jnp.int32)
    inp_rows, tgt_rows = _sc_gather(emb_u, emb_v, inp_idx, tgt_idx)
    inp_rows = inp_rows.reshape(B * L, D)
    tgt_rows = tgt_rows.reshape(B * T, D)
    tgt0 = target[:, :1].astype(jnp.int32)
    loss = _tc_loss(inp_rows, tgt_rows, tgt0)
    return loss.reshape(())
